# Initial kernel scaffold; baseline (speedup 1.0000x reference)
import jax
import jax.numpy as jnp
from jax.experimental import pallas as pl
from jax.experimental.pallas import tpu as pltpu

_B, _S, _H, _E, _TOPK = 4, 2048, 1024, 16, 2
_N = _B * _S
_BM = 512
_GRID = _N // _BM


def _router_kernel(x_ref, w1_ref, b1_ref, w2_ref, b2_ref,
                   idx_ref, p_ref, aux_ref, acc_ref):
    i = pl.program_id(0)
    h = jnp.dot(x_ref[...], w1_ref[...], preferred_element_type=jnp.float32,
                precision=jax.lax.Precision.HIGHEST)
    h = jnp.maximum(h + b1_ref[...], 0.0)
    logits = jnp.dot(h, w2_ref[...], preferred_element_type=jnp.float32,
                     precision=jax.lax.Precision.HIGHEST) + b2_ref[...]
    m = jnp.max(logits, axis=1, keepdims=True)
    e = jnp.exp(logits - m)
    s = jnp.sum(e, axis=1, keepdims=True)
    probs = e / s

    col = jax.lax.broadcasted_iota(jnp.int32, probs.shape, 1)
    p1 = jnp.max(probs, axis=1, keepdims=True)
    a1 = jnp.min(jnp.where(probs == p1, col, _E), axis=1, keepdims=True)
    masked = jnp.where(col == a1, -1.0, probs)
    p2 = jnp.max(masked, axis=1, keepdims=True)
    a2 = jnp.min(jnp.where(masked == p2, col, _E), axis=1, keepdims=True)
    tot = p1 + p2
    p_ref[...] = jnp.concatenate([p1 / tot, p2 / tot], axis=1)
    idx_ref[...] = jnp.concatenate([a1, a2], axis=1)

    @pl.when(i == 0)
    def _init():
        acc_ref[...] = jnp.zeros_like(acc_ref)

    acc_ref[...] += jnp.sum(probs, axis=0, keepdims=True)

    @pl.when(i == _GRID - 1)
    def _finish():
        mean = acc_ref[...] / _N
        aux_ref[...] = jnp.sum(mean * jnp.log(mean * _E + 1e-9),
                               keepdims=True).reshape(1, 1)


def kernel(x, W1, b1, W2, b2):
    x2 = x.reshape(_N, _H)
    idx, probs, aux = pl.pallas_call(
        _router_kernel,
        grid=(_GRID,),
        in_specs=[
            pl.BlockSpec((_BM, _H), lambda i: (i, 0)),
            pl.BlockSpec((_H, _H), lambda i: (0, 0)),
            pl.BlockSpec((1, _H), lambda i: (0, 0)),
            pl.BlockSpec((_H, _E), lambda i: (0, 0)),
            pl.BlockSpec((1, _E), lambda i: (0, 0)),
        ],
        out_specs=[
            pl.BlockSpec((_BM, _TOPK), lambda i: (i, 0)),
            pl.BlockSpec((_BM, _TOPK), lambda i: (i, 0)),
            pl.BlockSpec((1, 1), lambda i: (0, 0)),
        ],
        out_shape=[
            jax.ShapeDtypeStruct((_N, _TOPK), jnp.int32),
            jax.ShapeDtypeStruct((_N, _TOPK), jnp.float32),
            jax.ShapeDtypeStruct((1, 1), jnp.float32),
        ],
        scratch_shapes=[pltpu.VMEM((1, _E), jnp.float32)],
    )(x2, W1, b1.reshape(1, _H), W2, b2.reshape(1, _E))
    return (idx.reshape(_B, _S, _TOPK), probs.reshape(_B, _S, _TOPK),
            aux[0, 0])


# speedup vs baseline: 1.1110x; 1.1110x over previous
import jax
import jax.numpy as jnp
from jax.experimental import pallas as pl
from jax.experimental.pallas import tpu as pltpu

_B, _S, _H, _E, _TOPK = 4, 2048, 1024, 16, 2
_N = _B * _S
_BM = 512
_GRID = _N // _BM


def _router_kernel(x_ref, w1_ref, b1_ref, w2t_ref, b2_ref,
                   idx_ref, p_ref, aux_ref, acc_ref):
    i = pl.program_id(0)
    h = jnp.dot(x_ref[...], w1_ref[...], preferred_element_type=jnp.float32)
    h = jnp.maximum(h + b1_ref[...], 0.0)
    logits = jax.lax.dot_general(
        w2t_ref[...], h, (((1,), (1,)), ((), ())),
        preferred_element_type=jnp.float32) + b2_ref[...]

    row = jax.lax.broadcasted_iota(jnp.int32, logits.shape, 0)
    m = jnp.max(logits, axis=0, keepdims=True)
    a1 = jnp.min(jnp.where(logits == m, row, _E), axis=0, keepdims=True)
    e = jnp.exp(logits - m)
    s = jnp.sum(e, axis=0, keepdims=True)
    masked = jnp.where(row == a1, -1e30, logits)
    m2 = jnp.max(masked, axis=0, keepdims=True)
    a2 = jnp.min(jnp.where(masked == m2, row, _E), axis=0, keepdims=True)
    e2 = jnp.exp(m2 - m)
    rtot = 1.0 / (1.0 + e2)
    p_ref[...] = jnp.concatenate([rtot, e2 * rtot], axis=0)
    idx_ref[...] = jnp.concatenate([a1, a2], axis=0)

    @pl.when(i == 0)
    def _init():
        acc_ref[...] = jnp.zeros_like(acc_ref)

    acc_ref[...] += jnp.sum(e * (1.0 / s), axis=1, keepdims=True)

    @pl.when(i == _GRID - 1)
    def _finish():
        mean = acc_ref[...] / _N
        aux_ref[...] = jnp.sum(mean * jnp.log(mean * _E + 1e-9),
                               keepdims=True).reshape(1, 1)


def kernel(x, W1, b1, W2, b2):
    x2 = x.reshape(_N, _H)
    idx_t, probs_t, aux = pl.pallas_call(
        _router_kernel,
        grid=(_GRID,),
        in_specs=[
            pl.BlockSpec((_BM, _H), lambda i: (i, 0)),
            pl.BlockSpec((_H, _H), lambda i: (0, 0)),
            pl.BlockSpec((1, _H), lambda i: (0, 0)),
            pl.BlockSpec((_E, _H), lambda i: (0, 0)),
            pl.BlockSpec((_E, 1), lambda i: (0, 0)),
        ],
        out_specs=[
            pl.BlockSpec((_TOPK, _BM), lambda i: (0, i)),
            pl.BlockSpec((_TOPK, _BM), lambda i: (0, i)),
            pl.BlockSpec((1, 1), lambda i: (0, 0)),
        ],
        out_shape=[
            jax.ShapeDtypeStruct((_TOPK, _N), jnp.int32),
            jax.ShapeDtypeStruct((_TOPK, _N), jnp.float32),
            jax.ShapeDtypeStruct((1, 1), jnp.float32),
        ],
        scratch_shapes=[pltpu.VMEM((_E, 1), jnp.float32)],
    )(x2, W1, b1.reshape(1, _H), W2.T, b2.reshape(_E, 1))
    return (idx_t.T.reshape(_B, _S, _TOPK), probs_t.T.reshape(_B, _S, _TOPK),
            aux[0, 0])


# speedup vs baseline: 1.1726x; 1.0554x over previous
import jax
import jax.numpy as jnp
from jax.experimental import pallas as pl
from jax.experimental.pallas import tpu as pltpu

_B, _S, _H, _E, _TOPK = 4, 2048, 1024, 16, 2
_N = _B * _S
_SUB = 512
_NSUB = 2
_BM = _SUB * _NSUB
_GRID = _N // _BM


def _router_kernel(x_ref, w1_ref, b1_ref, w2t_ref, b2_ref,
                   idx_ref, p_ref, aux_ref, acc_ref):
    i = pl.program_id(0)

    @pl.when(i == 0)
    def _init():
        acc_ref[...] = jnp.zeros_like(acc_ref)

    for j in range(_NSUB):
        rows = pl.ds(j * _SUB, _SUB)
        h = jnp.dot(x_ref[rows, :], w1_ref[...],
                    preferred_element_type=jnp.float32)
        h = jnp.maximum(h + b1_ref[...], 0.0)
        logits = jax.lax.dot_general(
            w2t_ref[...], h, (((1,), (1,)), ((), ())),
            preferred_element_type=jnp.float32) + b2_ref[...]

        row = jax.lax.broadcasted_iota(jnp.int32, logits.shape, 0)
        m = jnp.max(logits, axis=0, keepdims=True)
        a1 = jnp.min(jnp.where(logits == m, row, _E), axis=0, keepdims=True)
        e = jnp.exp(logits - m)
        s = jnp.sum(e, axis=0, keepdims=True)
        masked = jnp.where(row == a1, -1e30, logits)
        m2 = jnp.max(masked, axis=0, keepdims=True)
        a2 = jnp.min(jnp.where(masked == m2, row, _E), axis=0, keepdims=True)
        e2 = jnp.exp(m2 - m)
        rtot = 1.0 / (1.0 + e2)
        cols = pl.ds(j * _SUB, _SUB)
        p_ref[:, cols] = jnp.concatenate([rtot, e2 * rtot], axis=0)
        idx_ref[:, cols] = jnp.concatenate([a1, a2], axis=0)
        acc_ref[...] += jnp.sum(e * (1.0 / s), axis=1, keepdims=True)

    @pl.when(i == _GRID - 1)
    def _finish():
        mean = acc_ref[...] / _N
        aux_ref[...] = jnp.sum(mean * jnp.log(mean * _E + 1e-9),
                               keepdims=True).reshape(1, 1)


def kernel(x, W1, b1, W2, b2):
    x2 = x.reshape(_N, _H)
    idx_t, probs_t, aux = pl.pallas_call(
        _router_kernel,
        grid=(_GRID,),
        in_specs=[
            pl.BlockSpec((_BM, _H), lambda i: (i, 0)),
            pl.BlockSpec((_H, _H), lambda i: (0, 0)),
            pl.BlockSpec((1, _H), lambda i: (0, 0)),
            pl.BlockSpec((_E, _H), lambda i: (0, 0)),
            pl.BlockSpec((_E, 1), lambda i: (0, 0)),
        ],
        out_specs=[
            pl.BlockSpec((_TOPK, _BM), lambda i: (0, i)),
            pl.BlockSpec((_TOPK, _BM), lambda i: (0, i)),
            pl.BlockSpec((1, 1), lambda i: (0, 0)),
        ],
        out_shape=[
            jax.ShapeDtypeStruct((_TOPK, _N), jnp.int32),
            jax.ShapeDtypeStruct((_TOPK, _N), jnp.float32),
            jax.ShapeDtypeStruct((1, 1), jnp.float32),
        ],
        scratch_shapes=[pltpu.VMEM((_E, 1), jnp.float32)],
    )(x2, W1, b1.reshape(1, _H), W2.T, b2.reshape(_E, 1))
    return (idx_t.T.reshape(_B, _S, _TOPK), probs_t.T.reshape(_B, _S, _TOPK),
            aux[0, 0])
